# baseline (speedup 1.0000x reference)
import functools

import jax
import jax.numpy as jnp
from jax import lax
from jax.experimental import pallas as pl
from jax.experimental.pallas import tpu as pltpu
from jax.experimental.pallas import tpu_sc as plsc

ROWS = 64
N = 32768
K = 256
L = 16
NVREG = N // L
NC = 2
NS = 16
ROWS_PER_W = ROWS // (NC * NS)
CAND_CAP = N + L
MASK7F = 0x7FFFFFFF


def _key(xv):
    i = lax.bitcast_convert_type(xv, jnp.int32)
    m = lax.shift_right_arithmetic(i, jnp.full((L,), 31, jnp.int32))
    return lax.bitwise_xor(i, lax.bitwise_and(m, _splat(MASK7F)))


def _splat(val):
    return jnp.full((L,), val, jnp.int32)


def _sload(ref, idx):
    v = plsc.load_gather(ref, [_splat(idx)])
    return jnp.max(v)


def _scount(mask):
    return jnp.max(plsc.all_reduce_population_count(mask))


def _zero_hist(hist):
    def body(c, _):
        hist[pl.ds(c * L, L)] = jnp.zeros((L,), jnp.int32)
        return 0

    lax.fori_loop(0, 256, body, 0)


def _crossing(hist, cum, k_rem, n_sub=L):
    def chunk(c, carry):
        acc = jnp.zeros((L,), jnp.int32)
        for lane in range(n_sub):
            acc = acc + hist[pl.ds(lane * 256 + c * L, L)]
        cs = plsc.cumsum(acc) + carry
        cum[pl.ds(c * L, L)] = cs
        return jnp.max(cs)

    n_act = lax.fori_loop(0, 256 // L, chunk, jnp.int32(0))
    target = n_act - k_rem

    def cnt(c, dacc):
        cs = cum[pl.ds(c * L, L)]
        return dacc + _scount(cs <= target)

    d = lax.fori_loop(0, 256 // L, cnt, jnp.int32(0))
    cum_d = _sload(cum, d)
    k_rem_new = k_rem - (n_act - cum_d)
    return d, k_rem_new


def _row_topk(xrow, cand_s, cand_i, hist, cum, eq_i, fk_s, fk_i, out_s,
              out_i, outv_v):
    hist2 = hist.at[pl.ds(4096, 4096)]
    lane = lax.iota(jnp.int32, L)
    ones = jnp.ones((L,), jnp.int32)
    tmask = jnp.ones((L,), jnp.bool_)

    _zero_hist(hist)
    _zero_hist(hist2)

    c4096 = _splat(4096)
    c24 = jnp.full((L,), 24, jnp.int32)

    def scan_a(v, off_v):
        s = _key(xrow[pl.ds(v * L, L)])
        d = lax.shift_right_arithmetic(s, c24) + 128
        plsc.addupdate_scatter(hist, [off_v + lane * 256 + d], ones,
                               mask=tmask)
        return c4096 - off_v

    lax.fori_loop(0, NVREG, scan_a, jnp.zeros((L,), jnp.int32))
    m0 = lane == 0
    d1, k_rem = _crossing(hist, cum, jnp.int32(K), n_sub=2 * L)
    t1 = lax.shift_left(d1 - 128, 24)
    t1v = _splat(t1)

    c16v = _splat(L)

    def scan_b(v, carry):
        cnt_v, idxv = carry
        s = _key(xrow[pl.ds(v * L, L)])
        m = s >= t1v
        posn = cnt_v * L + lane
        plsc.store_scatter(cand_s, [posn], s, mask=m)
        plsc.store_scatter(cand_i, [posn], idxv, mask=m)
        return cnt_v + m.astype(jnp.int32), idxv + c16v

    cnt_v, _ = lax.fori_loop(0, NVREG, scan_b,
                             (jnp.zeros((L,), jnp.int32), lane))
    nc_vregs = jnp.max(cnt_v)

    pfx = t1
    for lvl in range(3):
        sh = 16 - 8 * lvl
        hibits = 8 * (lvl + 1)
        _zero_hist(hist)
        pfx_v = _splat(pfx)
        shv = jnp.full((L,), sh, jnp.int32)
        hiv = jnp.full((L,), 32 - hibits, jnp.int32)

        def scan_l(v, _, pfx_v=pfx_v, shv=shv, hiv=hiv):
            s = cand_s[pl.ds(v * L, L)]
            val = _splat(v) < cnt_v
            act = (lax.shift_right_logical(lax.bitwise_xor(s, pfx_v), hiv)
                   == 0) & val
            d = lax.bitwise_and(
                lax.shift_right_arithmetic(s, shv), _splat(0xFF))
            plsc.addupdate_scatter(hist, [lane * 256 + d],
                                   act.astype(jnp.int32), mask=tmask)
            return 0

        lax.fori_loop(0, nc_vregs, scan_l, 0)
        d_l, k_rem = _crossing(hist, cum, k_rem)
        pfx = lax.bitwise_or(pfx, lax.shift_left(d_l, sh))

    s_star = pfx
    r_star = k_rem
    s_star_v = _splat(s_star)

    def scan_f(v, carry):
        fpos, epos = carry
        s = cand_s[pl.ds(v * L, L)]
        iv = cand_i[pl.ds(v * L, L)]
        val = _splat(v) < cnt_v
        m_gt = (s > s_star_v) & val
        m_eq = (s == s_star_v) & val
        plsc.store_compressed(fk_s.at[pl.ds(fpos, L)], s, mask=m_gt)
        plsc.store_compressed(fk_i.at[pl.ds(fpos, L)], iv, mask=m_gt)
        plsc.store_compressed(eq_i.at[pl.ds(epos, L)], iv, mask=m_eq)
        return fpos + _scount(m_gt), epos + _scount(m_eq)

    fpos, n_eq = lax.fori_loop(0, nc_vregs, scan_f,
                               (jnp.int32(0), jnp.int32(0)))
    fpos_v = _splat(fpos)
    rstar_v = _splat(r_star)
    neq_v = _splat(n_eq)
    ev = (n_eq + L - 1) // L

    def eq_body(e, _):
        ie_v = _splat(_sload(eq_i, e))

        def eq_inner(r, cnt):
            iv = eq_i[pl.ds(r * L, L)]
            val = (_splat(r * L) + lane) < neq_v
            return cnt + ((iv < ie_v) & val).astype(jnp.int32)

        cnt = lax.fori_loop(0, ev, eq_inner, jnp.zeros((L,), jnp.int32))
        rank_v = _splat(jnp.sum(cnt))
        keep = m0 & (rank_v < rstar_v)
        plsc.store_scatter(fk_s, [fpos_v + rank_v], s_star_v, mask=keep)
        plsc.store_scatter(fk_i, [fpos_v + rank_v], ie_v, mask=keep)
        return 0

    lax.fori_loop(0, n_eq, eq_body, 0)


    def rank_body(i, _):
        si = _sload(fk_s, i)
        ii = _sload(fk_i, i)
        si_v = _splat(si)
        ii_v = _splat(ii)

        def inner(j, cnt):
            s = fk_s[pl.ds(j * L, L)]
            idx = fk_i[pl.ds(j * L, L)]
            c = (s > si_v) | ((s == si_v) & (idx < ii_v))
            return cnt + c.astype(jnp.int32)

        cnt = lax.fori_loop(0, K // L, inner, jnp.zeros((L,), jnp.int32))
        rank = jnp.sum(cnt)
        rv = _splat(rank)
        plsc.store_scatter(out_s, [rv], si_v, mask=m0)
        plsc.store_scatter(out_i, [rv], ii_v, mask=m0)
        return 0

    lax.fori_loop(0, K, rank_body, 0)

    def outconv(c, _):
        s = out_s[pl.ds(c * L, L)]
        m = lax.shift_right_arithmetic(s, jnp.full((L,), 31, jnp.int32))
        i = lax.bitwise_xor(s, lax.bitwise_and(m, _splat(MASK7F)))
        outv_v[pl.ds(c * L, L)] = lax.bitcast_convert_type(i, jnp.float32) * 2.0
        return 0

    lax.fori_loop(0, K // L, outconv, 0)


def _make_kernel():
    mesh = plsc.VectorSubcoreMesh(core_axis_name="c", subcore_axis_name="s")

    @functools.partial(
        pl.kernel,
        out_type=(
            jax.ShapeDtypeStruct((ROWS, K), jnp.float32),
            jax.ShapeDtypeStruct((ROWS, K), jnp.int32),
        ),
        mesh=mesh,
        compiler_params=pltpu.CompilerParams(needs_layout_passes=False),
        scratch_types=[
            pltpu.VMEM((N,), jnp.float32),
            pltpu.VMEM((CAND_CAP,), jnp.int32),
            pltpu.VMEM((CAND_CAP,), jnp.int32),
            pltpu.VMEM((8192,), jnp.int32),
            pltpu.VMEM((256,), jnp.int32),
            pltpu.VMEM((4096,), jnp.int32),
            pltpu.VMEM((K + L,), jnp.int32),
            pltpu.VMEM((K + L,), jnp.int32),
            pltpu.VMEM((K,), jnp.int32),
            pltpu.VMEM((K,), jnp.int32),
            pltpu.VMEM((K,), jnp.float32),
        ],
    )
    def topk_kernel(x_hbm, outv_hbm, outi_hbm, xrow, cand_s, cand_i, hist,
                    cum, eq_i, fk_s, fk_i, out_s, out_i, outv_v):
        wid = lax.axis_index("s") * NC + lax.axis_index("c")
        for r in range(ROWS_PER_W):
            row = wid * ROWS_PER_W + r
            pltpu.sync_copy(x_hbm.at[row], xrow)
            _row_topk(xrow, cand_s, cand_i, hist, cum, eq_i, fk_s, fk_i,
                      out_s, out_i, outv_v)
            pltpu.sync_copy(outv_v, outv_hbm.at[row])
            pltpu.sync_copy(out_i, outi_hbm.at[row])

    return topk_kernel


_topk = _make_kernel()


@jax.jit
def kernel(tensor):
    values, indices = _topk(tensor)
    return (values, indices)


# speedup vs baseline: 1.1718x; 1.1718x over previous
import functools

import jax
import jax.numpy as jnp
from jax import lax
from jax.experimental import pallas as pl
from jax.experimental.pallas import tpu as pltpu
from jax.experimental.pallas import tpu_sc as plsc

ROWS = 64
N = 32768
K = 256
L = 16
NVREG = N // L
NC = 2
NS = 16
ROWS_PER_W = ROWS // (NC * NS)
CAND_CAP = N + L
MASK7F = 0x7FFFFFFF


def _key(xv):
    i = lax.bitcast_convert_type(xv, jnp.int32)
    m = lax.shift_right_arithmetic(i, jnp.full((L,), 31, jnp.int32))
    return lax.bitwise_xor(i, lax.bitwise_and(m, _splat(MASK7F)))


def _splat(val):
    return jnp.full((L,), val, jnp.int32)


def _sload(ref, idx):
    v = plsc.load_gather(ref, [_splat(idx)])
    return jnp.max(v)


def _scount(mask):
    return jnp.max(plsc.all_reduce_population_count(mask))


def _zero_hist(hist):
    def body(c, _):
        hist[pl.ds(c * L, L)] = jnp.zeros((L,), jnp.int32)
        return 0

    lax.fori_loop(0, 256, body, 0)


def _crossing(hist, cum, k_rem):
    def chunk(c, carry):
        acc = jnp.zeros((L,), jnp.int32)
        for lane in range(L):
            acc = acc + hist[pl.ds(lane * 256 + c * L, L)]
        cs = plsc.cumsum(acc) + carry
        cum[pl.ds(c * L, L)] = cs
        return jnp.max(cs)

    n_act = lax.fori_loop(0, 256 // L, chunk, jnp.int32(0))
    target = n_act - k_rem

    def cnt(c, dacc):
        cs = cum[pl.ds(c * L, L)]
        return dacc + _scount(cs <= target)

    d = lax.fori_loop(0, 256 // L, cnt, jnp.int32(0))
    cum_d = _sload(cum, d)
    k_rem_new = k_rem - (n_act - cum_d)
    return d, k_rem_new


def _row_topk(xrow, cand_s, cand_i, hist, cum, eq_i, fk_s, fk_i, out_s,
              out_i, outv_v):
    lane = lax.iota(jnp.int32, L)
    ones = jnp.ones((L,), jnp.int32)
    tmask = jnp.ones((L,), jnp.bool_)

    _zero_hist(hist)

    def scan_a(v, _):
        s = _key(xrow[pl.ds(v * L, L)])
        d = lax.shift_right_arithmetic(s, jnp.full((L,), 24, jnp.int32)) + 128
        plsc.addupdate_scatter(hist, [lane * 256 + d], ones, mask=tmask)
        return 0

    lax.fori_loop(0, NVREG, scan_a, 0)
    m0 = lane == 0
    d1, k_rem = _crossing(hist, cum, jnp.int32(K))
    t1 = lax.shift_left(d1 - 128, 24)
    t1v = _splat(t1)

    c16v = _splat(L)

    def scan_b(v, carry):
        cnt_v, idxv = carry
        s = _key(xrow[pl.ds(v * L, L)])
        m = s >= t1v
        posn = cnt_v * L + lane
        plsc.store_scatter(cand_s, [posn], s, mask=m)
        plsc.store_scatter(cand_i, [posn], idxv, mask=m)
        return cnt_v + m.astype(jnp.int32), idxv + c16v

    cnt_v, _ = lax.fori_loop(0, NVREG, scan_b,
                             (jnp.zeros((L,), jnp.int32), lane))
    nc_vregs = jnp.max(cnt_v)

    pfx = t1
    for lvl in range(3):
        sh = 16 - 8 * lvl
        hibits = 8 * (lvl + 1)
        _zero_hist(hist)
        pfx_v = _splat(pfx)
        shv = jnp.full((L,), sh, jnp.int32)
        hiv = jnp.full((L,), 32 - hibits, jnp.int32)

        def scan_l(v, _, pfx_v=pfx_v, shv=shv, hiv=hiv):
            s = cand_s[pl.ds(v * L, L)]
            val = _splat(v) < cnt_v
            act = (lax.shift_right_logical(lax.bitwise_xor(s, pfx_v), hiv)
                   == 0) & val
            d = lax.bitwise_and(
                lax.shift_right_arithmetic(s, shv), _splat(0xFF))
            plsc.addupdate_scatter(hist, [lane * 256 + d],
                                   act.astype(jnp.int32), mask=tmask)
            return 0

        lax.fori_loop(0, nc_vregs, scan_l, 0)
        d_l, k_rem = _crossing(hist, cum, k_rem)
        pfx = lax.bitwise_or(pfx, lax.shift_left(d_l, sh))

    s_star = pfx
    r_star = k_rem
    s_star_v = _splat(s_star)

    def scan_f(v, carry):
        fpos, epos = carry
        s = cand_s[pl.ds(v * L, L)]
        iv = cand_i[pl.ds(v * L, L)]
        val = _splat(v) < cnt_v
        m_gt = (s > s_star_v) & val
        m_eq = (s == s_star_v) & val
        plsc.store_compressed(fk_s.at[pl.ds(fpos, L)], s, mask=m_gt)
        plsc.store_compressed(fk_i.at[pl.ds(fpos, L)], iv, mask=m_gt)
        plsc.store_compressed(eq_i.at[pl.ds(epos, L)], iv, mask=m_eq)
        return fpos + _scount(m_gt), epos + _scount(m_eq)

    fpos, n_eq = lax.fori_loop(0, nc_vregs, scan_f,
                               (jnp.int32(0), jnp.int32(0)))
    fpos_v = _splat(fpos)
    rstar_v = _splat(r_star)
    neq_v = _splat(n_eq)
    ev = (n_eq + L - 1) // L

    def eq_body(e, _):
        ie_v = _splat(_sload(eq_i, e))

        def eq_inner(r, cnt):
            iv = eq_i[pl.ds(r * L, L)]
            val = (_splat(r * L) + lane) < neq_v
            return cnt + ((iv < ie_v) & val).astype(jnp.int32)

        cnt = lax.fori_loop(0, ev, eq_inner, jnp.zeros((L,), jnp.int32))
        rank_v = _splat(jnp.sum(cnt))
        keep = m0 & (rank_v < rstar_v)
        plsc.store_scatter(fk_s, [fpos_v + rank_v], s_star_v, mask=keep)
        plsc.store_scatter(fk_i, [fpos_v + rank_v], ie_v, mask=keep)
        return 0

    lax.fori_loop(0, n_eq, eq_body, 0)


    zv = jnp.zeros((L,), jnp.int32)

    def rank_body(h, _):
        i0 = h * 2
        sa_v = _splat(_sload(fk_s, i0))
        ia_v = _splat(_sload(fk_i, i0))
        sb_v = _splat(_sload(fk_s, i0 + 1))
        ib_v = _splat(_sload(fk_i, i0 + 1))

        def inner(j, carry):
            ca, cb = carry
            s = fk_s[pl.ds(j * L, L)]
            idx = fk_i[pl.ds(j * L, L)]
            ca = ca + ((s > sa_v)
                       | ((s == sa_v) & (idx < ia_v))).astype(jnp.int32)
            cb = cb + ((s > sb_v)
                       | ((s == sb_v) & (idx < ib_v))).astype(jnp.int32)
            return ca, cb

        ca, cb = lax.fori_loop(0, K // L, inner, (zv, zv))
        ra_v = _splat(jnp.sum(ca))
        rb_v = _splat(jnp.sum(cb))
        plsc.store_scatter(out_s, [ra_v], sa_v, mask=m0)
        plsc.store_scatter(out_i, [ra_v], ia_v, mask=m0)
        plsc.store_scatter(out_s, [rb_v], sb_v, mask=m0)
        plsc.store_scatter(out_i, [rb_v], ib_v, mask=m0)
        return 0

    lax.fori_loop(0, K // 2, rank_body, 0)

    def outconv(c, _):
        s = out_s[pl.ds(c * L, L)]
        m = lax.shift_right_arithmetic(s, jnp.full((L,), 31, jnp.int32))
        i = lax.bitwise_xor(s, lax.bitwise_and(m, _splat(MASK7F)))
        outv_v[pl.ds(c * L, L)] = lax.bitcast_convert_type(i, jnp.float32) * 2.0
        return 0

    lax.fori_loop(0, K // L, outconv, 0)


def _make_kernel():
    mesh = plsc.VectorSubcoreMesh(core_axis_name="c", subcore_axis_name="s")

    @functools.partial(
        pl.kernel,
        out_type=(
            jax.ShapeDtypeStruct((ROWS, K), jnp.float32),
            jax.ShapeDtypeStruct((ROWS, K), jnp.int32),
        ),
        mesh=mesh,
        compiler_params=pltpu.CompilerParams(needs_layout_passes=False),
        scratch_types=[
            pltpu.VMEM((N,), jnp.float32),
            pltpu.VMEM((CAND_CAP,), jnp.int32),
            pltpu.VMEM((CAND_CAP,), jnp.int32),
            pltpu.VMEM((4096,), jnp.int32),
            pltpu.VMEM((256,), jnp.int32),
            pltpu.VMEM((4096,), jnp.int32),
            pltpu.VMEM((K + L,), jnp.int32),
            pltpu.VMEM((K + L,), jnp.int32),
            pltpu.VMEM((K,), jnp.int32),
            pltpu.VMEM((K,), jnp.int32),
            pltpu.VMEM((K,), jnp.float32),
        ],
    )
    def topk_kernel(x_hbm, outv_hbm, outi_hbm, xrow, cand_s, cand_i, hist,
                    cum, eq_i, fk_s, fk_i, out_s, out_i, outv_v):
        wid = lax.axis_index("s") * NC + lax.axis_index("c")
        for r in range(ROWS_PER_W):
            row = wid * ROWS_PER_W + r
            pltpu.sync_copy(x_hbm.at[row], xrow)
            _row_topk(xrow, cand_s, cand_i, hist, cum, eq_i, fk_s, fk_i,
                      out_s, out_i, outv_v)
            pltpu.sync_copy(outv_v, outv_hbm.at[row])
            pltpu.sync_copy(out_i, outi_hbm.at[row])

    return topk_kernel


_topk = _make_kernel()


@jax.jit
def kernel(tensor):
    values, indices = _topk(tensor)
    return (values, indices)


# speedup vs baseline: 1.2219x; 1.0427x over previous
import functools

import jax
import jax.numpy as jnp
from jax import lax
from jax.experimental import pallas as pl
from jax.experimental.pallas import tpu as pltpu
from jax.experimental.pallas import tpu_sc as plsc

ROWS = 64
N = 32768
K = 256
L = 16
NVREG = N // L
NC = 2
NS = 16
ROWS_PER_W = ROWS // (NC * NS)
CAND_CAP = N + L
MASK7F = 0x7FFFFFFF


def _key(xv):
    i = lax.bitcast_convert_type(xv, jnp.int32)
    m = lax.shift_right_arithmetic(i, jnp.full((L,), 31, jnp.int32))
    return lax.bitwise_xor(i, lax.bitwise_and(m, _splat(MASK7F)))


def _splat(val):
    return jnp.full((L,), val, jnp.int32)


def _sload(ref, idx):
    v = plsc.load_gather(ref, [_splat(idx)])
    return jnp.max(v)


def _scount(mask):
    return jnp.max(plsc.all_reduce_population_count(mask))


def _zero_hist(hist):
    def body(c, _):
        hist[pl.ds(c * L, L)] = jnp.zeros((L,), jnp.int32)
        return 0

    lax.fori_loop(0, 256, body, 0)


def _crossing(hist, cum, k_rem):
    def chunk(c, carry):
        acc = jnp.zeros((L,), jnp.int32)
        for lane in range(L):
            acc = acc + hist[pl.ds(lane * 256 + c * L, L)]
        cs = plsc.cumsum(acc) + carry
        cum[pl.ds(c * L, L)] = cs
        return jnp.max(cs)

    n_act = lax.fori_loop(0, 256 // L, chunk, jnp.int32(0))
    target = n_act - k_rem

    def cnt(c, dacc):
        cs = cum[pl.ds(c * L, L)]
        return dacc + _scount(cs <= target)

    d = lax.fori_loop(0, 256 // L, cnt, jnp.int32(0))
    cum_d = _sload(cum, d)
    k_rem_new = k_rem - (n_act - cum_d)
    return d, k_rem_new


def _row_topk(xrow, cand_s, cand_i, hist, cum, eq_i, fk_s, fk_i, out_s,
              out_i, outv_v):
    lane = lax.iota(jnp.int32, L)
    ones = jnp.ones((L,), jnp.int32)
    tmask = jnp.ones((L,), jnp.bool_)

    _zero_hist(hist)

    def scan_a(v, _):
        s = _key(xrow[pl.ds(v * L, L)])
        d = lax.shift_right_arithmetic(s, jnp.full((L,), 24, jnp.int32)) + 128
        plsc.addupdate_scatter(hist, [lane * 256 + d], ones, mask=tmask)
        return 0

    lax.fori_loop(0, NVREG, scan_a, 0)
    m0 = lane == 0
    d1, k_rem = _crossing(hist, cum, jnp.int32(K))
    t1 = lax.shift_left(d1 - 128, 24)
    t1v = _splat(t1)

    c16v = _splat(L)

    def scan_b(v, carry):
        cnt_v, idxv = carry
        s = _key(xrow[pl.ds(v * L, L)])
        m = s >= t1v
        posn = cnt_v * L + lane
        plsc.store_scatter(cand_s, [posn], s, mask=m)
        plsc.store_scatter(cand_i, [posn], idxv, mask=m)
        return cnt_v + m.astype(jnp.int32), idxv + c16v

    cnt_v, _ = lax.fori_loop(0, NVREG, scan_b,
                             (jnp.zeros((L,), jnp.int32), lane))
    nc_vregs = jnp.max(cnt_v)

    pfx = t1
    for lvl in range(3):
        sh = 16 - 8 * lvl
        hibits = 8 * (lvl + 1)
        _zero_hist(hist)
        pfx_v = _splat(pfx)
        shv = jnp.full((L,), sh, jnp.int32)
        hiv = jnp.full((L,), 32 - hibits, jnp.int32)

        def scan_l(v, _, pfx_v=pfx_v, shv=shv, hiv=hiv):
            s = cand_s[pl.ds(v * L, L)]
            val = _splat(v) < cnt_v
            act = (lax.shift_right_logical(lax.bitwise_xor(s, pfx_v), hiv)
                   == 0) & val
            d = lax.bitwise_and(
                lax.shift_right_arithmetic(s, shv), _splat(0xFF))
            plsc.addupdate_scatter(hist, [lane * 256 + d],
                                   act.astype(jnp.int32), mask=tmask)
            return 0

        lax.fori_loop(0, nc_vregs, scan_l, 0)
        d_l, k_rem = _crossing(hist, cum, k_rem)
        pfx = lax.bitwise_or(pfx, lax.shift_left(d_l, sh))

    s_star = pfx
    r_star = k_rem
    s_star_v = _splat(s_star)

    def scan_f(v, carry):
        fpos, epos = carry
        s = cand_s[pl.ds(v * L, L)]
        iv = cand_i[pl.ds(v * L, L)]
        val = _splat(v) < cnt_v
        m_gt = (s > s_star_v) & val
        m_eq = (s == s_star_v) & val
        plsc.store_compressed(fk_s.at[pl.ds(fpos, L)], s, mask=m_gt)
        plsc.store_compressed(fk_i.at[pl.ds(fpos, L)], iv, mask=m_gt)
        plsc.store_compressed(eq_i.at[pl.ds(epos, L)], iv, mask=m_eq)
        return fpos + _scount(m_gt), epos + _scount(m_eq)

    fpos, n_eq = lax.fori_loop(0, nc_vregs, scan_f,
                               (jnp.int32(0), jnp.int32(0)))
    fpos_v = _splat(fpos)
    rstar_v = _splat(r_star)
    neq_v = _splat(n_eq)
    ev = (n_eq + L - 1) // L

    def eq_body(e, _):
        ie_v = _splat(_sload(eq_i, e))

        def eq_inner(r, cnt):
            iv = eq_i[pl.ds(r * L, L)]
            val = (_splat(r * L) + lane) < neq_v
            return cnt + ((iv < ie_v) & val).astype(jnp.int32)

        cnt = lax.fori_loop(0, ev, eq_inner, jnp.zeros((L,), jnp.int32))
        rank_v = _splat(jnp.sum(cnt))
        keep = m0 & (rank_v < rstar_v)
        plsc.store_scatter(fk_s, [fpos_v + rank_v], s_star_v, mask=keep)
        plsc.store_scatter(fk_i, [fpos_v + rank_v], ie_v, mask=keep)
        return 0

    lax.fori_loop(0, n_eq, eq_body, 0)


    zv = jnp.zeros((L,), jnp.int32)

    def rank_body(h, _):
        i0 = h * 4
        sv = [_splat(_sload(fk_s, i0 + t)) for t in range(4)]
        iv = [_splat(_sload(fk_i, i0 + t)) for t in range(4)]

        def inner(j, carry):
            s = fk_s[pl.ds(j * L, L)]
            idx = fk_i[pl.ds(j * L, L)]
            return tuple(
                c + ((s > sv[t])
                     | ((s == sv[t]) & (idx < iv[t]))).astype(jnp.int32)
                for t, c in enumerate(carry))

        cnts = lax.fori_loop(0, K // L, inner, (zv, zv, zv, zv))
        for t in range(4):
            rv = _splat(jnp.sum(cnts[t]))
            plsc.store_scatter(out_s, [rv], sv[t], mask=m0)
            plsc.store_scatter(out_i, [rv], iv[t], mask=m0)
        return 0

    lax.fori_loop(0, K // 4, rank_body, 0)

    def outconv(c, _):
        s = out_s[pl.ds(c * L, L)]
        m = lax.shift_right_arithmetic(s, jnp.full((L,), 31, jnp.int32))
        i = lax.bitwise_xor(s, lax.bitwise_and(m, _splat(MASK7F)))
        outv_v[pl.ds(c * L, L)] = lax.bitcast_convert_type(i, jnp.float32) * 2.0
        return 0

    lax.fori_loop(0, K // L, outconv, 0)


def _make_kernel():
    mesh = plsc.VectorSubcoreMesh(core_axis_name="c", subcore_axis_name="s")

    @functools.partial(
        pl.kernel,
        out_type=(
            jax.ShapeDtypeStruct((ROWS, K), jnp.float32),
            jax.ShapeDtypeStruct((ROWS, K), jnp.int32),
        ),
        mesh=mesh,
        compiler_params=pltpu.CompilerParams(needs_layout_passes=False),
        scratch_types=[
            pltpu.VMEM((N,), jnp.float32),
            pltpu.VMEM((CAND_CAP,), jnp.int32),
            pltpu.VMEM((CAND_CAP,), jnp.int32),
            pltpu.VMEM((4096,), jnp.int32),
            pltpu.VMEM((256,), jnp.int32),
            pltpu.VMEM((4096,), jnp.int32),
            pltpu.VMEM((K + L,), jnp.int32),
            pltpu.VMEM((K + L,), jnp.int32),
            pltpu.VMEM((K,), jnp.int32),
            pltpu.VMEM((K,), jnp.int32),
            pltpu.VMEM((K,), jnp.float32),
        ],
    )
    def topk_kernel(x_hbm, outv_hbm, outi_hbm, xrow, cand_s, cand_i, hist,
                    cum, eq_i, fk_s, fk_i, out_s, out_i, outv_v):
        wid = lax.axis_index("s") * NC + lax.axis_index("c")
        for r in range(ROWS_PER_W):
            row = wid * ROWS_PER_W + r
            pltpu.sync_copy(x_hbm.at[row], xrow)
            _row_topk(xrow, cand_s, cand_i, hist, cum, eq_i, fk_s, fk_i,
                      out_s, out_i, outv_v)
            pltpu.sync_copy(outv_v, outv_hbm.at[row])
            pltpu.sync_copy(out_i, outi_hbm.at[row])

    return topk_kernel


_topk = _make_kernel()


@jax.jit
def kernel(tensor):
    values, indices = _topk(tensor)
    return (values, indices)
